# Initial kernel scaffold; baseline (speedup 1.0000x reference)
import functools

import jax
import jax.numpy as jnp
from jax import lax
from jax.experimental import pallas as pl
from jax.experimental.pallas import tpu as pltpu
from jax.experimental.pallas import tpu_sc as plsc

N_IN = 100000
N_OUT = 50000
K = 8
E = 12500
C = 32

E_PAD = 12800
TOT = K * E_PAD
NC, NS = 2, 16
NW = NC * NS
BPW = TOT // NW
IBLK = 128
NBLK = BPW // IBLK
SH = 51200
ZROWS = SH // NS
ZB = 160
DUM = N_OUT

_mesh = plsc.VectorSubcoreMesh(core_axis_name="c", subcore_axis_name="s",
                               num_cores=NC, num_subcores=NS)


@functools.partial(
    pl.kernel,
    out_type=jax.ShapeDtypeStruct((TOT, C), jnp.float32),
    mesh=_mesh,
    scratch_types=[
        pltpu.VMEM((NBLK, IBLK), jnp.int32),
        pltpu.VMEM((BPW, C), jnp.float32),
        pltpu.SemaphoreType.DMA,
    ],
)
def _gather_stage(feats_hbm, idx_hbm, out_hbm, idx_v, rows_v, sem):
    wid = lax.axis_index("s") * NC + lax.axis_index("c")
    pltpu.sync_copy(idx_hbm.at[wid], idx_v)

    @pl.loop(0, NBLK)
    def _fire(j):
        pltpu.async_copy(feats_hbm.at[idx_v.at[j]],
                         rows_v.at[pl.ds(j * IBLK, IBLK)], sem)

    @pl.loop(0, NBLK)
    def _drain(j):
        pltpu.make_async_copy(feats_hbm.at[idx_v.at[j]],
                              rows_v.at[pl.ds(j * IBLK, IBLK)], sem).wait()

    pltpu.sync_copy(rows_v, out_hbm.at[pl.ds(wid * BPW, BPW)])


def _mm_body(g_ref, w_ref, o_ref):
    o_ref[0] = jnp.dot(g_ref[0], w_ref[0], preferred_element_type=jnp.float32)


def _matmul_stage(gathered, W):
    return pl.pallas_call(
        _mm_body,
        grid=(K,),
        in_specs=[pl.BlockSpec((1, E_PAD, C), lambda k: (k, 0, 0)),
                  pl.BlockSpec((1, C, C), lambda k: (k, 0, 0))],
        out_specs=pl.BlockSpec((1, E_PAD, C), lambda k: (k, 0, 0)),
        out_shape=jax.ShapeDtypeStruct((K, E_PAD, C), jnp.float32),
    )(gathered.reshape(K, E_PAD, C), W)


@functools.partial(
    pl.kernel,
    out_type=jax.ShapeDtypeStruct((NC, SH, C), jnp.float32),
    mesh=_mesh,
    scratch_types=[
        pltpu.VMEM_SHARED((SH, C), jnp.float32),
        pltpu.VMEM((NBLK, IBLK), jnp.int32),
        pltpu.VMEM((BPW, C), jnp.float32),
        pltpu.VMEM((ZB, C), jnp.float32),
        pltpu.SemaphoreType.DMA,
    ],
)
def _scatter_stage(msg_hbm, idx_hbm, part_hbm, acc_sh, idx_v, rows_v, zb_v, sem):
    cid = lax.axis_index("c")
    sid = lax.axis_index("s")
    wid = sid * NC + cid

    @pl.loop(0, ZB)
    def _zrow(i):
        zb_v[i, pl.ds(0, 16)] = jnp.zeros((16,), jnp.float32)
        zb_v[i, pl.ds(16, 16)] = jnp.zeros((16,), jnp.float32)

    @pl.loop(0, ZROWS // ZB)
    def _zcopy(z):
        pltpu.sync_copy(zb_v, acc_sh.at[pl.ds(sid * ZROWS + z * ZB, ZB)])

    plsc.subcore_barrier()

    pltpu.sync_copy(idx_hbm.at[wid], idx_v)
    pltpu.sync_copy(msg_hbm.at[pl.ds(wid * BPW, BPW)], rows_v)

    @pl.loop(0, NBLK)
    def _scat(j):
        pltpu.sync_copy(rows_v.at[pl.ds(j * IBLK, IBLK)],
                        acc_sh.at[idx_v.at[j]], add=True)

    plsc.subcore_barrier()

    pltpu.sync_copy(acc_sh.at[pl.ds(sid * ZROWS, ZROWS)],
                    part_hbm.at[cid].at[pl.ds(sid * ZROWS, ZROWS)])


DBLK = 2000


def _add_body(p_ref, o_ref):
    o_ref[...] = p_ref[0] + p_ref[1]


def _reduce_stage(partials):
    return pl.pallas_call(
        _add_body,
        grid=(N_OUT // DBLK,),
        in_specs=[pl.BlockSpec((NC, DBLK, C), lambda i: (0, i, 0))],
        out_specs=pl.BlockSpec((DBLK, C), lambda i: (i, 0)),
        out_shape=jax.ShapeDtypeStruct((N_OUT, C), jnp.float32),
    )(partials)


def kernel(feats, in_indices, out_indices, W):
    pad = E_PAD - E
    gpad = (jnp.arange(pad, dtype=jnp.int32) * 37) % N_IN
    spad = DUM + (jnp.arange(pad, dtype=jnp.int32) % (SH - DUM))
    in_p = jnp.concatenate(
        [in_indices, jnp.broadcast_to(gpad, (K, pad))], axis=1)
    out_p = jnp.concatenate(
        [out_indices, jnp.broadcast_to(spad, (K, pad))], axis=1)
    in_arr = in_p.reshape(NW, NBLK, IBLK)
    out_arr = out_p.reshape(NW, NBLK, IBLK)

    gathered = _gather_stage(feats, in_arr)
    msg = _matmul_stage(gathered, W).reshape(TOT, C)
    partials = _scatter_stage(msg, out_arr)
    return _reduce_stage(partials)


# speedup vs baseline: 3.8707x; 3.8707x over previous
import functools

import jax
import jax.numpy as jnp
from jax import lax
from jax.experimental import pallas as pl
from jax.experimental.pallas import tpu as pltpu
from jax.experimental.pallas import tpu_sc as plsc

N_IN = 100000
N_OUT = 50000
K = 8
E = 12500
C = 32

E_PAD = 12800
TOT = K * E_PAD
NC, NS = 2, 16
NW = NC * NS
BPW = TOT // NW
IBLK = 128
NBLK = BPW // IBLK
OWN = N_OUT // NC
SHH = 25600
ZROWS = SHH // NS
ZB = 160
R = 1280
CH = TOT // NS // R
NBLK2 = R // IBLK

_mesh = plsc.VectorSubcoreMesh(core_axis_name="c", subcore_axis_name="s",
                               num_cores=NC, num_subcores=NS)
_sc_params = pltpu.CompilerParams(use_tc_tiling_on_sc=False)


@functools.partial(
    pl.kernel,
    out_type=jax.ShapeDtypeStruct((TOT, C), jnp.float32),
    mesh=_mesh,
    scratch_types=[
        pltpu.VMEM((NBLK, IBLK), jnp.int32),
        pltpu.VMEM((BPW, C), jnp.float32),
        pltpu.SemaphoreType.DMA,
    ],
    compiler_params=_sc_params,
)
def _gather_stage(feats_hbm, idx_hbm, out_hbm, idx_v, rows_v, sem):
    wid = lax.axis_index("s") * NC + lax.axis_index("c")
    pltpu.sync_copy(idx_hbm.at[wid], idx_v)

    @pl.loop(0, NBLK)
    def _fire(j):
        pltpu.async_copy(feats_hbm.at[idx_v.at[j]],
                         rows_v.at[pl.ds(j * IBLK, IBLK)], sem)

    @pl.loop(0, NBLK)
    def _drain(j):
        pltpu.make_async_copy(feats_hbm.at[idx_v.at[j]],
                              rows_v.at[pl.ds(j * IBLK, IBLK)], sem).wait()

    pltpu.sync_copy(rows_v, out_hbm.at[pl.ds(wid * BPW, BPW)])


def _mm_body(g_ref, w_ref, o_ref):
    o_ref[0] = jnp.dot(g_ref[0], w_ref[0], preferred_element_type=jnp.float32)


def _matmul_stage(gathered, W):
    return pl.pallas_call(
        _mm_body,
        grid=(K,),
        in_specs=[pl.BlockSpec((1, E_PAD, C), lambda k: (k, 0, 0)),
                  pl.BlockSpec((1, C, C), lambda k: (k, 0, 0))],
        out_specs=pl.BlockSpec((1, E_PAD, C), lambda k: (k, 0, 0)),
        out_shape=jax.ShapeDtypeStruct((K, E_PAD, C), jnp.float32),
    )(gathered.reshape(K, E_PAD, C), W)


@functools.partial(
    pl.kernel,
    out_type=jax.ShapeDtypeStruct((NC * SHH, C), jnp.float32),
    mesh=_mesh,
    scratch_types=[
        pltpu.VMEM_SHARED((SHH, C), jnp.float32),
        pltpu.VMEM((NBLK2, IBLK), jnp.int32),
        pltpu.VMEM((R, C), jnp.float32),
        pltpu.VMEM((ZB, C), jnp.float32),
        pltpu.SemaphoreType.DMA,
    ],
    compiler_params=_sc_params,
)
def _scatter_stage(msg_hbm, idx_hbm, part_hbm, acc_sh, idx_v, rows_v,
                   zb_v, sem):
    cid = lax.axis_index("c")
    sid = lax.axis_index("s")
    lo = cid * OWN

    @pl.loop(0, ZB)
    def _zrow(i):
        zb_v[i, pl.ds(0, 16)] = jnp.zeros((16,), jnp.float32)
        zb_v[i, pl.ds(16, 16)] = jnp.zeros((16,), jnp.float32)

    @pl.loop(0, ZROWS // ZB)
    def _zcopy(z):
        pltpu.sync_copy(zb_v, acc_sh.at[pl.ds(sid * ZROWS + z * ZB, ZB)])

    plsc.subcore_barrier()

    @pl.loop(0, CH)
    def _chunk(q):
        pltpu.sync_copy(idx_hbm.at[sid, q], idx_v)
        pltpu.sync_copy(msg_hbm.at[pl.ds(sid * (CH * R) + q * R, R)], rows_v)

        @pl.loop(0, NBLK2)
        def _route(j):
            @pl.loop(0, IBLK // 16)
            def _vec(t):
                v = idx_v[j, pl.ds(t * 16, 16)]
                owned = (v >= lo) & (v < lo + OWN)
                dummy = OWN + (v & 511)
                idx_v[j, pl.ds(t * 16, 16)] = jnp.where(owned, v - lo, dummy)

        @pl.loop(0, NBLK2)
        def _scat(j):
            pltpu.sync_copy(rows_v.at[pl.ds(j * IBLK, IBLK)],
                            acc_sh.at[idx_v.at[j]], add=True)

    plsc.subcore_barrier()

    pltpu.sync_copy(acc_sh.at[pl.ds(sid * ZROWS, ZROWS)],
                    part_hbm.at[pl.ds(cid * SHH + sid * ZROWS, ZROWS)])


def kernel(feats, in_indices, out_indices, W):
    pad = E_PAD - E
    gpad = (jnp.arange(pad, dtype=jnp.int32) * 37) % N_IN
    spad = (1 << 20) + (jnp.arange(pad, dtype=jnp.int32) % 512)
    in_p = jnp.concatenate(
        [in_indices, jnp.broadcast_to(gpad, (K, pad))], axis=1)
    out_p = jnp.concatenate(
        [out_indices, jnp.broadcast_to(spad, (K, pad))], axis=1)
    in_arr = in_p.reshape(NW, NBLK, IBLK)
    out_arr = out_p.reshape(NS, CH, NBLK2, IBLK)

    gathered = _gather_stage(feats, in_arr)
    msg = _matmul_stage(gathered, W).reshape(TOT, C)
    partials = _scatter_stage(msg, out_arr)
    return partials.reshape(NC, SHH, C)[:, :OWN, :].reshape(N_OUT, C)


# speedup vs baseline: 6.7018x; 1.7314x over previous
import functools

import jax
import jax.numpy as jnp
from jax import lax
from jax.experimental import pallas as pl
from jax.experimental.pallas import tpu as pltpu
from jax.experimental.pallas import tpu_sc as plsc

N_IN = 100000
N_OUT = 50000
K = 8
E = 12500
C = 32

E_PAD = 12800
TOT = K * E_PAD
NC, NS = 2, 16
NW = NC * NS
BPW = TOT // NW
IBLK = 128
NBLK = BPW // IBLK
OWN = N_OUT // NC
SHH = 25600
ZROWS = SHH // NS
ZB = 160
R = 1280
CH = TOT // NS // R
NBLK2 = R // IBLK

_mesh = plsc.VectorSubcoreMesh(core_axis_name="c", subcore_axis_name="s",
                               num_cores=NC, num_subcores=NS)
_sc_params = pltpu.CompilerParams(use_tc_tiling_on_sc=False)


@functools.partial(
    pl.kernel,
    out_type=jax.ShapeDtypeStruct((TOT, C), jnp.float32),
    mesh=_mesh,
    scratch_types=[
        pltpu.VMEM((NBLK, IBLK), jnp.int32),
        pltpu.VMEM((BPW, C), jnp.float32),
        pltpu.SemaphoreType.DMA,
    ],
    compiler_params=_sc_params,
)
def _gather_stage(feats_hbm, idx_hbm, out_hbm, idx_v, rows_v, sem):
    wid = lax.axis_index("s") * NC + lax.axis_index("c")
    pltpu.sync_copy(idx_hbm.at[wid], idx_v)

    @pl.loop(0, NBLK)
    def _fire(j):
        pltpu.async_copy(feats_hbm.at[idx_v.at[j]],
                         rows_v.at[pl.ds(j * IBLK, IBLK)], sem)

    @pl.loop(0, NBLK)
    def _drain(j):
        pltpu.make_async_copy(feats_hbm.at[idx_v.at[j]],
                              rows_v.at[pl.ds(j * IBLK, IBLK)], sem).wait()

    pltpu.sync_copy(rows_v, out_hbm.at[pl.ds(wid * BPW, BPW)])


def _mm_body(g_ref, w_ref, o_ref):
    o_ref[...] = jnp.dot(g_ref[...], w_ref[0],
                         preferred_element_type=jnp.float32)


def _matmul_stage(gathered, Wb):
    g128 = gathered.reshape(TOT // 4, 4 * C)
    msg128 = pl.pallas_call(
        _mm_body,
        grid=(K,),
        in_specs=[pl.BlockSpec((E_PAD // 4, 4 * C), lambda k: (k, 0)),
                  pl.BlockSpec((1, 4 * C, 4 * C), lambda k: (k, 0, 0))],
        out_specs=pl.BlockSpec((E_PAD // 4, 4 * C), lambda k: (k, 0)),
        out_shape=jax.ShapeDtypeStruct((TOT // 4, 4 * C), jnp.float32),
    )(g128, Wb)
    return msg128.reshape(TOT, C)


@functools.partial(
    pl.kernel,
    out_type=jax.ShapeDtypeStruct((N_OUT, C), jnp.float32),
    mesh=_mesh,
    scratch_types=[
        pltpu.VMEM_SHARED((SHH, C), jnp.float32),
        pltpu.VMEM((NBLK2, IBLK), jnp.int32),
        pltpu.VMEM((R, C), jnp.float32),
        pltpu.VMEM((ZB, C), jnp.float32),
        pltpu.SemaphoreType.DMA,
    ],
    compiler_params=_sc_params,
)
def _scatter_stage(msg_hbm, idx_hbm, part_hbm, acc_sh, idx_v, rows_v,
                   zb_v, sem):
    cid = lax.axis_index("c")
    sid = lax.axis_index("s")
    lo = cid * OWN

    @pl.loop(0, ZB)
    def _zrow(i):
        zb_v[i, pl.ds(0, 16)] = jnp.zeros((16,), jnp.float32)
        zb_v[i, pl.ds(16, 16)] = jnp.zeros((16,), jnp.float32)

    @pl.loop(0, ZROWS // ZB)
    def _zfire(z):
        pltpu.async_copy(zb_v, acc_sh.at[pl.ds(sid * ZROWS + z * ZB, ZB)],
                         sem)

    @pl.loop(0, ZROWS // ZB)
    def _zdrain(z):
        pltpu.make_async_copy(zb_v,
                              acc_sh.at[pl.ds(sid * ZROWS + z * ZB, ZB)],
                              sem).wait()

    plsc.subcore_barrier()

    @pl.loop(0, CH)
    def _chunk(q):
        pltpu.sync_copy(idx_hbm.at[sid, q], idx_v)
        pltpu.sync_copy(msg_hbm.at[pl.ds(sid * (CH * R) + q * R, R)], rows_v)

        @pl.loop(0, NBLK2)
        def _route(j):
            @pl.loop(0, IBLK // 16)
            def _vec(t):
                v = idx_v[j, pl.ds(t * 16, 16)]
                owned = (v >= lo) & (v < lo + OWN)
                dummy = OWN + (v & 511)
                idx_v[j, pl.ds(t * 16, 16)] = jnp.where(owned, v - lo, dummy)

        @pl.loop(0, NBLK2)
        def _sfire(j):
            pltpu.async_copy(rows_v.at[pl.ds(j * IBLK, IBLK)],
                             acc_sh.at[idx_v.at[j]], sem, add=True)

        @pl.loop(0, NBLK2)
        def _sdrain(j):
            pltpu.make_async_copy(rows_v.at[pl.ds(j * IBLK, IBLK)],
                                  acc_sh.at[idx_v.at[j]], sem).wait()

    plsc.subcore_barrier()

    @pl.when(sid < NS - 1)
    def _wb_full():
        pltpu.sync_copy(acc_sh.at[pl.ds(sid * ZROWS, ZROWS)],
                        part_hbm.at[pl.ds(cid * OWN + sid * ZROWS, ZROWS)])

    @pl.when(sid == NS - 1)
    def _wb_tail():
        pltpu.sync_copy(acc_sh.at[pl.ds(sid * ZROWS, OWN - (NS - 1) * ZROWS)],
                        part_hbm.at[pl.ds(cid * OWN + sid * ZROWS,
                                          OWN - (NS - 1) * ZROWS)])


def kernel(feats, in_indices, out_indices, W):
    pad = E_PAD - E
    gpad = (jnp.arange(pad, dtype=jnp.int32) * 37) % N_IN
    spad = (1 << 20) + (jnp.arange(pad, dtype=jnp.int32) % 512)
    in_p = jnp.concatenate(
        [in_indices, jnp.broadcast_to(gpad, (K, pad))], axis=1)
    out_p = jnp.concatenate(
        [out_indices, jnp.broadcast_to(spad, (K, pad))], axis=1)
    in_arr = in_p.reshape(NW, NBLK, IBLK)
    out_arr = out_p.reshape(NS, CH, NBLK2, IBLK)

    Wb = jnp.einsum('ij,kab->kiajb', jnp.eye(4, dtype=W.dtype),
                    W).reshape(K, 4 * C, 4 * C)

    gathered = _gather_stage(feats, in_arr)
    msg = _matmul_stage(gathered, Wb)
    return _scatter_stage(msg, out_arr)
